# baseline (device time: 96306 ns/iter reference)
import jax
import jax.numpy as jnp
from jax import lax
from jax.experimental import pallas as pl
from jax.experimental.pallas import tpu as pltpu

N_DEV = 16
NA = 9
NB = 8


def kernel(x, Wq, K_ext, V_ext, Wo):
    B, Sq, Din = x.shape
    _, HD = Wq.shape
    Bg, Skv, Hq, Dh = K_ext.shape
    Hloc = HD // Dh
    Dout = Wo.shape[1]
    BSq = B * Sq
    bf16 = jnp.bfloat16

    pos = lax.axis_index("i")
    k2 = lax.dynamic_slice_in_dim(K_ext, pos * B, B, axis=0).astype(
        bf16).reshape(B * Skv, Hq * Dh)
    v2 = lax.dynamic_slice_in_dim(V_ext, pos * B, B, axis=0).astype(
        bf16).reshape(B * Skv, Hq * Dh)

    def body(x_ref, wq_ref, k_ref, v_ref, wo_ref, out_ref,
             aq, ao, bq, bo, ctx_s, acc,
             aq_s, aq_r, ao_s, ao_r, bq_s, bq_r, bo_s, bo_r):
        my = lax.axis_index("i")
        left = lax.rem(my + N_DEV - 1, N_DEV)
        right = lax.rem(my + 1, N_DEV)

        wqb = wq_ref[...].astype(bf16)
        wob = wo_ref[...].astype(bf16)
        aq[0, :, :] = wqb
        ao[0, :, :] = wob
        bq[0, :, :] = wqb
        bo[0, :, :] = wob

        bar = pltpu.get_barrier_semaphore()
        for nbr in (left, right):
            pl.semaphore_signal(
                bar, inc=1, device_id=(nbr,),
                device_id_type=pl.DeviceIdType.MESH)
        pl.semaphore_wait(bar, 2)

        x2 = x_ref[...].reshape(BSq, Din).astype(bf16)
        acc[...] = jnp.zeros((BSq, Dout), jnp.float32)

        def mk_aq(h):
            return pltpu.make_async_remote_copy(
                src_ref=aq.at[h], dst_ref=aq.at[h + 1],
                send_sem=aq_s.at[h], recv_sem=aq_r.at[h],
                device_id=(right,), device_id_type=pl.DeviceIdType.MESH)

        def mk_ao(h):
            return pltpu.make_async_remote_copy(
                src_ref=ao.at[h], dst_ref=ao.at[h + 1],
                send_sem=ao_s.at[h], recv_sem=ao_r.at[h],
                device_id=(right,), device_id_type=pl.DeviceIdType.MESH)

        def mk_bq(h):
            return pltpu.make_async_remote_copy(
                src_ref=bq.at[h], dst_ref=bq.at[h + 1],
                send_sem=bq_s.at[h], recv_sem=bq_r.at[h],
                device_id=(left,), device_id_type=pl.DeviceIdType.MESH)

        def mk_bo(h):
            return pltpu.make_async_remote_copy(
                src_ref=bo.at[h], dst_ref=bo.at[h + 1],
                send_sem=bo_s.at[h], recv_sem=bo_r.at[h],
                device_id=(left,), device_id_type=pl.DeviceIdType.MESH)

        def compute(wq_k, wo_k, jj):
            q = jnp.dot(x2, wq_k, preferred_element_type=jnp.float32)
            for b in range(B):
                kb = k_ref[b * Skv:(b + 1) * Skv, pl.ds(jj * HD, HD)]
                vb = v_ref[b * Skv:(b + 1) * Skv, pl.ds(jj * HD, HD)]
                qb = q[b * Sq:(b + 1) * Sq, :].astype(bf16)
                for hh in range(Hloc):
                    qh = qb[:, hh * Dh:(hh + 1) * Dh]
                    kh = kb[:, hh * Dh:(hh + 1) * Dh]
                    vh = vb[:, hh * Dh:(hh + 1) * Dh]
                    s = lax.dot_general(
                        qh, kh, (((1,), (1,)), ((), ())),
                        preferred_element_type=jnp.float32)
                    w = jnp.exp(s * 0.125)
                    w = (w / jnp.sum(w, axis=-1, keepdims=True)).astype(bf16)
                    ctx_s[b * Sq:(b + 1) * Sq, hh * Dh:(hh + 1) * Dh] = (
                        jnp.dot(w, vh,
                                preferred_element_type=jnp.float32)
                        .astype(bf16))
            acc[...] = acc[...] + jnp.dot(
                ctx_s[...], wo_k, preferred_element_type=jnp.float32)

        def compute_a(h):
            compute(aq[pl.ds(h, 1)].reshape(Din, HD),
                    ao[pl.ds(h, 1)].reshape(HD, Dout),
                    lax.rem(my - h + N_DEV, N_DEV))

        def compute_b(h):
            compute(bq[pl.ds(h, 1)].reshape(Din, HD),
                    bo[pl.ds(h, 1)].reshape(HD, Dout),
                    lax.rem(my + h, N_DEV))

        mk_aq(0).start()
        mk_ao(0).start()
        mk_bq(0).start()
        mk_bo(0).start()
        compute_a(0)

        def hop(h, carry):
            mk_aq(h - 1).wait_recv()
            mk_aq(h).start()
            mk_ao(h - 1).wait_recv()
            mk_ao(h).start()

            @pl.when(h < NB - 1)
            def _():
                mk_bq(h - 1).wait_recv()
                mk_bq(h).start()
                mk_bo(h - 1).wait_recv()
                mk_bo(h).start()

            compute_a(h)

            @pl.when(h < NB - 1)
            def _():
                compute_b(h)
            return carry

        lax.fori_loop(1, NA - 1, hop, None)

        mk_bq(NB - 2).wait_recv()
        mk_bo(NB - 2).wait_recv()
        compute_b(NB - 1)
        mk_aq(NA - 2).wait_recv()
        mk_ao(NA - 2).wait_recv()
        compute_a(NA - 1)

        for h in range(NA - 1):
            mk_aq(h).wait_send()
            mk_ao(h).wait_send()
        for h in range(NB - 1):
            mk_bq(h).wait_send()
            mk_bo(h).wait_send()

        out_ref[...] = acc[...].reshape(B, Sq, Dout)

    return pl.pallas_call(
        body,
        out_shape=jax.ShapeDtypeStruct((B, Sq, Dout), jnp.float32),
        in_specs=[
            pl.BlockSpec(memory_space=pltpu.VMEM),
            pl.BlockSpec(memory_space=pltpu.VMEM),
            pl.BlockSpec(memory_space=pltpu.VMEM),
            pl.BlockSpec(memory_space=pltpu.VMEM),
            pl.BlockSpec(memory_space=pltpu.VMEM),
        ],
        out_specs=pl.BlockSpec(memory_space=pltpu.VMEM),
        scratch_shapes=[
            pltpu.VMEM((NA, Din, HD), bf16),
            pltpu.VMEM((NA, HD, Dout), bf16),
            pltpu.VMEM((NB, Din, HD), bf16),
            pltpu.VMEM((NB, HD, Dout), bf16),
            pltpu.VMEM((BSq, HD), bf16),
            pltpu.VMEM((BSq, Dout), jnp.float32),
            pltpu.SemaphoreType.DMA((NA - 1,)),
            pltpu.SemaphoreType.DMA((NA - 1,)),
            pltpu.SemaphoreType.DMA((NA - 1,)),
            pltpu.SemaphoreType.DMA((NA - 1,)),
            pltpu.SemaphoreType.DMA((NB - 1,)),
            pltpu.SemaphoreType.DMA((NB - 1,)),
            pltpu.SemaphoreType.DMA((NB - 1,)),
            pltpu.SemaphoreType.DMA((NB - 1,)),
        ],
        compiler_params=pltpu.CompilerParams(
            collective_id=0, vmem_limit_bytes=56 * 1024 * 1024),
    )(x, Wq, k2, v2, Wo)
